# baseline (speedup 1.0000x reference)
import jax
import jax.numpy as jnp
from jax import lax
from jax.experimental import pallas as pl
from jax.experimental.pallas import tpu as pltpu
from jax.experimental.pallas import tpu_sc as plsc

_B = 4096
_ND = 13
_NS = 26
_V = 100000
_D = 16
_REP = 64
_H1, _H2 = 256, 128
_NC, _NSUB = 2, 16
_NW = _NC * _NSUB
_RPW = _B * _NS // _NW


def _sc_gather_body(idx_hbm, e2_hbm, e1_hbm, rows_out, scal_out,
                    idx_v, rows_v, scal_v, sem_r, sem_s):
    wid = lax.axis_index("s") * _NC + lax.axis_index("c")
    base = wid * _RPW
    pltpu.sync_copy(idx_hbm.at[pl.ds(base, _RPW)], idx_v)
    cp_r = pltpu.async_copy(e2_hbm.at[idx_v], rows_v, sem_r)
    cp_s = pltpu.async_copy(e1_hbm.at[idx_v], scal_v, sem_s)
    cp_r.wait()
    cp_s.wait()
    pltpu.sync_copy(rows_v, rows_out.at[pl.ds(base, _RPW)])
    pltpu.sync_copy(scal_v, scal_out.at[pl.ds(base, _RPW)])


def _sc_gather(flat_idx, e2f, e1f):
    return pl.kernel(
        _sc_gather_body,
        out_type=(jax.ShapeDtypeStruct((_B * _NS, _D), jnp.float32),
                  jax.ShapeDtypeStruct((_B * _NS,), jnp.float32)),
        mesh=plsc.VectorSubcoreMesh(core_axis_name="c", subcore_axis_name="s"),
        scratch_types=[pltpu.VMEM((_RPW,), jnp.int32),
                       pltpu.VMEM((_RPW, _D), jnp.float32),
                       pltpu.VMEM((_RPW,), jnp.float32),
                       pltpu.SemaphoreType.DMA,
                       pltpu.SemaphoreType.DMA],
        compiler_params=pltpu.CompilerParams(use_tc_tiling_on_sc=False),
    )(flat_idx, e2f, e1f)

_BLK = 512


def _tc_body(xg_ref, xd_ref, rep_ref, e1g_ref,
             w1a_ref, w1b_ref, w1c_ref, bd1_ref,
             wd2_ref, bd2_ref, wf_ref,
             w1da_ref, w1db_ref, cb_ref,
             out_ref):
    f32 = jnp.float32

    def dot(a, b):
        return lax.dot_general(a, b, (((1,), (0,)), ((), ())),
                               preferred_element_type=f32)

    xg = xg_ref[...]
    xd = xd_ref[...]
    rp = rep_ref[...]
    h1 = dot(xg, w1a_ref[...]) + dot(xd, w1b_ref[...]) + dot(rp, w1c_ref[...])
    h1 = jnp.maximum(h1 + bd1_ref[...], 0.0)
    h2 = jnp.maximum(dot(h1, wd2_ref[...]) + bd2_ref[...], 0.0)
    dnn = dot(h2, wf_ref[...])
    fm1d = dot(xd, w1da_ref[...]) + dot(rp, w1db_ref[...])
    r = lax.broadcasted_iota(jnp.int32, (_NS * _D, _D), 0)
    c = lax.broadcasted_iota(jnp.int32, (_NS * _D, _D), 1)
    m = ((r % _D) == c).astype(f32)
    s = dot(xg, m)
    ssq = dot(xg * xg, m)
    fm2 = 0.5 * jnp.sum(s * s - ssq, axis=1, keepdims=True)
    fm1s = jnp.sum(e1g_ref[...], axis=1, keepdims=True)
    out_ref[...] = dnn + fm1d + fm2 + fm1s + cb_ref[...]


def _tc_dense(xg, xd, rep, e1g, w1a, w1b, w1c, bd1, wd2, bd2, wf, w1da, w1db, cb):
    def blk(shape):
        return pl.BlockSpec(shape, lambda i: (i, 0))

    def full(shape):
        return pl.BlockSpec(shape, lambda i: (0, 0))

    return pl.pallas_call(
        _tc_body,
        grid=(_B // _BLK,),
        in_specs=[blk((_BLK, _NS * _D)), blk((_BLK, _ND)), blk((_BLK, _REP)),
                  blk((_BLK, _NS)),
                  full((_NS * _D, _H1)), full((_ND, _H1)), full((_REP, _H1)),
                  full((1, _H1)),
                  full((_H1, _H2)), full((1, _H2)), full((_H2, 1)),
                  full((_ND, 1)), full((_REP, 1)), full((1, 1))],
        out_specs=blk((_BLK, 1)),
        out_shape=jax.ShapeDtypeStruct((_B, 1), jnp.float32),
    )(xg, xd, rep, e1g, w1a, w1b, w1c, bd1, wd2, bd2, wf, w1da, w1db, cb)


def kernel(representation, target_x, e1, e2, W1d, b1d, Wd1, bd1, Wd2, bd2, Wf, bf):
    sparse_idx = target_x[:, _ND:].astype(jnp.int32)
    flat_idx = (sparse_idx
                + (jnp.arange(_NS, dtype=jnp.int32) * _V)[None, :]).reshape(-1)
    rows, scal = _sc_gather(flat_idx, e2.reshape(_NS * _V, _D),
                            e1.reshape(_NS * _V))
    xg = rows.reshape(_B, _NS * _D)
    e1g = scal.reshape(_B, _NS)
    xd = target_x[:, :_ND]
    out = _tc_dense(
        xg, xd, representation, e1g,
        Wd1[:_NS * _D], Wd1[_NS * _D:_NS * _D + _ND], Wd1[_NS * _D + _ND:],
        bd1.reshape(1, _H1), Wd2, bd2.reshape(1, _H2), Wf,
        W1d[:_ND], W1d[_ND:], (b1d + bf).reshape(1, 1))
    return out
